# Initial kernel scaffold; baseline (speedup 1.0000x reference)
import functools

import jax
import jax.numpy as jnp
from jax import lax
from jax.experimental import pallas as pl
from jax.experimental.pallas import tpu as pltpu
from jax.experimental.pallas import tpu_sc as plsc

N_NODES = 10000
N_EDGES = 320000
NC, NS = 2, 16
NW = NC * NS
E_W = N_EDGES // NW
K = 128
N_FULL = E_W // K
TAIL = E_W - N_FULL * K
ROWS_TILE = N_NODES // NS


def _feat_scatter(W):
  mesh = plsc.VectorSubcoreMesh(core_axis_name="c", subcore_axis_name="s")

  @functools.partial(
      pl.kernel,
      out_type=jax.ShapeDtypeStruct((NC, N_NODES, W), jnp.float32),
      mesh=mesh,
      scratch_types=[
          pltpu.VMEM((K,), jnp.int32),
          pltpu.VMEM((K,), jnp.int32),
          pltpu.VMEM((K, W), jnp.float32),
          pltpu.VMEM((TAIL,), jnp.int32),
          pltpu.VMEM((TAIL,), jnp.int32),
          pltpu.VMEM((TAIL, W), jnp.float32),
          pltpu.VMEM_SHARED((N_NODES, W), jnp.float32),
          pltpu.SemaphoreType.DMA,
      ],
  )
  def k(y_hbm, src_hbm, dst_hbm, zeros_hbm, out_hbm,
        src_v, dst_v, rows_v, srct_v, dstt_v, rowst_v, acc, sem):
    cid = lax.axis_index("c")
    sid = lax.axis_index("s")
    e0 = (cid * NS + sid) * E_W

    pltpu.sync_copy(zeros_hbm, acc.at[pl.ds(sid * ROWS_TILE, ROWS_TILE)])
    plsc.subcore_barrier()

    def body(j, carry):
      base = e0 + j * K
      pltpu.sync_copy(src_hbm.at[pl.ds(base, K)], src_v)
      pltpu.sync_copy(dst_hbm.at[pl.ds(base, K)], dst_v)
      pltpu.async_copy(y_hbm.at[src_v], rows_v, sem).wait()
      pltpu.sync_copy(rows_v, acc.at[dst_v], add=True)
      return carry

    lax.fori_loop(0, N_FULL, body, 0)

    tbase = e0 + N_FULL * K
    pltpu.sync_copy(src_hbm.at[pl.ds(tbase, TAIL)], srct_v)
    pltpu.sync_copy(dst_hbm.at[pl.ds(tbase, TAIL)], dstt_v)
    pltpu.async_copy(y_hbm.at[srct_v], rowst_v, sem).wait()
    pltpu.sync_copy(rowst_v, acc.at[dstt_v], add=True)

    plsc.subcore_barrier()
    pltpu.sync_copy(acc.at[pl.ds(sid * ROWS_TILE, ROWS_TILE)],
                    out_hbm.at[cid, pl.ds(sid * ROWS_TILE, ROWS_TILE)])

  return k


def _count_scatter():
  W = 16
  mesh = plsc.VectorSubcoreMesh(core_axis_name="c", subcore_axis_name="s")

  @functools.partial(
      pl.kernel,
      out_type=jax.ShapeDtypeStruct((NC, N_NODES, W), jnp.float32),
      mesh=mesh,
      scratch_types=[
          pltpu.VMEM((K,), jnp.int32),
          pltpu.VMEM((K, W), jnp.float32),
          pltpu.VMEM((TAIL,), jnp.int32),
          pltpu.VMEM_SHARED((N_NODES, W), jnp.float32),
      ],
  )
  def k(dst_hbm, ones_hbm, zeros_hbm, out_hbm, dst_v, ones_v, dstt_v, acc):
    cid = lax.axis_index("c")
    sid = lax.axis_index("s")
    e0 = (cid * NS + sid) * E_W

    pltpu.sync_copy(zeros_hbm, acc.at[pl.ds(sid * ROWS_TILE, ROWS_TILE)])
    pltpu.sync_copy(ones_hbm, ones_v)
    plsc.subcore_barrier()

    def body(j, carry):
      base = e0 + j * K
      pltpu.sync_copy(dst_hbm.at[pl.ds(base, K)], dst_v)
      pltpu.sync_copy(ones_v, acc.at[dst_v], add=True)
      return carry

    lax.fori_loop(0, N_FULL, body, 0)

    tbase = e0 + N_FULL * K
    pltpu.sync_copy(dst_hbm.at[pl.ds(tbase, TAIL)], dstt_v)
    pltpu.sync_copy(ones_v.at[pl.ds(0, TAIL)], acc.at[dstt_v], add=True)

    plsc.subcore_barrier()
    pltpu.sync_copy(acc.at[pl.ds(sid * ROWS_TILE, ROWS_TILE)],
                    out_hbm.at[cid, pl.ds(sid * ROWS_TILE, ROWS_TILE)])

  return k


def _bn(h, gamma, beta, eps=1e-5):
  m = jnp.mean(h, axis=0, keepdims=True)
  v = jnp.mean((h - m) ** 2, axis=0, keepdims=True)
  return gamma * (h - m) * jax.lax.rsqrt(v + eps) + beta


def _k0_body(x_ref, wl_ref, wr_ref, b_ref, y_ref, z_ref):
  x = x_ref[...]
  y_ref[...] = jnp.dot(x, wl_ref[...], preferred_element_type=jnp.float32)
  z_ref[...] = jnp.dot(x, wr_ref[...], preferred_element_type=jnp.float32) + b_ref[...]


def _k0b_body(s_ref, c_ref, z_ref, g_ref, be_ref, h_ref, inv_ref):
  cnt = c_ref[0, :, 0:1] + c_ref[1, :, 0:1]
  inv = 1.0 / jnp.maximum(cnt, 1.0)
  inv_ref[...] = jnp.broadcast_to(inv, inv_ref.shape)
  mean = (s_ref[0] + s_ref[1]) * inv
  h = mean + z_ref[...]
  h_ref[...] = jnp.maximum(_bn(h, g_ref[...], be_ref[...]), 0.0)


def _mid_body(h_ref, s_ref, inv_ref, wl_ref, wr_ref, b_ref, g_ref, be_ref, o_ref):
  inv = inv_ref[:, 0:1]
  mean = (s_ref[0] + s_ref[1]) * inv
  h = (jnp.dot(mean, wl_ref[...], preferred_element_type=jnp.float32)
       + jnp.dot(h_ref[...], wr_ref[...], preferred_element_type=jnp.float32)
       + b_ref[...])
  o_ref[...] = jnp.maximum(_bn(h, g_ref[...], be_ref[...]), 0.0)


def _wide_body(h_ref, sa_ref, sb_ref, inv_ref, wla_ref, wlb_ref, wr_ref,
               b_ref, g_ref, be_ref, o_ref):
  inv = inv_ref[:, 0:1]
  mean_a = (sa_ref[0] + sa_ref[1]) * inv
  mean_b = (sb_ref[0] + sb_ref[1]) * inv
  h = (jnp.dot(mean_a, wla_ref[...], preferred_element_type=jnp.float32)
       + jnp.dot(mean_b, wlb_ref[...], preferred_element_type=jnp.float32)
       + jnp.dot(h_ref[...], wr_ref[...], preferred_element_type=jnp.float32)
       + b_ref[...])
  o_ref[...] = jnp.maximum(_bn(h, g_ref[...], be_ref[...]), 0.0)


def _last_body(h_ref, sa_ref, sb_ref, inv_ref, wla_ref, wlb_ref, wr_ref,
               b_ref, o_ref):
  inv = inv_ref[:, 0:1]
  mean_a = (sa_ref[0] + sa_ref[1]) * inv
  mean_b = (sb_ref[0] + sb_ref[1]) * inv
  o_ref[...] = (jnp.dot(mean_a, wla_ref[...], preferred_element_type=jnp.float32)
                + jnp.dot(mean_b, wlb_ref[...], preferred_element_type=jnp.float32)
                + jnp.dot(h_ref[...], wr_ref[...], preferred_element_type=jnp.float32)
                + b_ref[...])


def kernel(x, edge_index, params):
  n = x.shape[0]
  src = edge_index[0]
  dst = edge_index[1]
  f32 = jnp.float32

  zeros64 = jnp.zeros((ROWS_TILE, 64), f32)
  zeros128 = jnp.zeros((ROWS_TILE, 128), f32)
  zeros16 = jnp.zeros((ROWS_TILE, 16), f32)
  ones16 = jnp.ones((K, 16), f32)

  scat64 = _feat_scatter(64)
  scat128 = _feat_scatter(128)
  cnt_k = _count_scatter()

  sds = jax.ShapeDtypeStruct
  cparts = cnt_k(dst, ones16, zeros16)

  y0, z0 = pl.pallas_call(
      _k0_body,
      out_shape=(sds((n, 64), f32), sds((n, 64), f32)),
  )(x, params["Wl0"], params["Wr0"], params["b0"].reshape(1, 64))
  s0 = scat64(y0, src, dst, zeros64)
  h1, inv = pl.pallas_call(
      _k0b_body,
      out_shape=(sds((n, 64), f32), sds((n, 16), f32)),
  )(s0, cparts, z0, params["gamma0"].reshape(1, 64), params["beta0"].reshape(1, 64))

  s1 = scat64(h1, src, dst, zeros64)
  h2 = pl.pallas_call(
      _mid_body,
      out_shape=sds((n, 128), f32),
  )(h1, s1, inv, params["Wl1"], params["Wr1"], params["b1"].reshape(1, 128),
    params["gamma1"].reshape(1, 128), params["beta1"].reshape(1, 128))

  s2 = scat128(h2, src, dst, zeros128)
  h3 = pl.pallas_call(
      _mid_body,
      out_shape=sds((n, 256), f32),
  )(h2, s2, inv, params["Wl2"], params["Wr2"], params["b2"].reshape(1, 256),
    params["gamma2"].reshape(1, 256), params["beta2"].reshape(1, 256))

  s3a = scat128(h3[:, :128], src, dst, zeros128)
  s3b = scat128(h3[:, 128:], src, dst, zeros128)
  h4 = pl.pallas_call(
      _wide_body,
      out_shape=sds((n, 256), f32),
  )(h3, s3a, s3b, inv, params["Wl3"][:128], params["Wl3"][128:],
    params["Wr3"], params["b3"].reshape(1, 256),
    params["gamma3"].reshape(1, 256), params["beta3"].reshape(1, 256))

  s4a = scat128(h4[:, :128], src, dst, zeros128)
  s4b = scat128(h4[:, 128:], src, dst, zeros128)
  out = pl.pallas_call(
      _last_body,
      out_shape=sds((n, 576), f32),
  )(h4, s4a, s4b, inv, params["Wl4"][:128], params["Wl4"][128:],
    params["Wr4"], params["b4"].reshape(1, 576))
  return out


# speedup vs baseline: 5.1017x; 5.1017x over previous
import functools

import jax
import jax.numpy as jnp
from jax import lax
from jax.experimental import pallas as pl
from jax.experimental.pallas import tpu as pltpu
from jax.experimental.pallas import tpu_sc as plsc

N_NODES = 10000
N_EDGES = 320000
NC, NS = 2, 16
NW = NC * NS
K = 128
N_CHUNKS = N_EDGES // K
W = 128

ROWS_TILE = 632
ROWS_LAST = N_NODES - (NS - 1) * ROWS_TILE


def _zero_or_dump(sid, zeros_hbm, acc, out_ref=None, cid=None):
  off = sid * ROWS_TILE

  @pl.when(sid < NS - 1)
  def _():
    if out_ref is None:
      pltpu.sync_copy(zeros_hbm, acc.at[pl.ds(off, ROWS_TILE)])
    else:
      pltpu.sync_copy(acc.at[pl.ds(off, ROWS_TILE)],
                      out_ref.at[cid, pl.ds(off, ROWS_TILE)])

  @pl.when(sid == NS - 1)
  def _():
    if out_ref is None:
      pltpu.sync_copy(zeros_hbm.at[pl.ds(0, ROWS_LAST)],
                      acc.at[pl.ds(off, ROWS_LAST)])
    else:
      pltpu.sync_copy(acc.at[pl.ds(off, ROWS_LAST)],
                      out_ref.at[cid, pl.ds(off, ROWS_LAST)])


def _make_feat_scatter():
  mesh = plsc.VectorSubcoreMesh(core_axis_name="c", subcore_axis_name="s")

  @functools.partial(
      pl.kernel,
      out_type=jax.ShapeDtypeStruct((NC, N_NODES, W), jnp.float32),
      mesh=mesh,
      scratch_types=[
          pltpu.VMEM((K,), jnp.int32),
          pltpu.VMEM((K,), jnp.int32),
          pltpu.VMEM((K, W), jnp.float32),
          pltpu.VMEM_SHARED((N_NODES, W), jnp.float32),
          pltpu.SemaphoreType.DMA,
      ],
  )
  def k(y_hbm, src_hbm, dst_hbm, zeros_hbm, out_hbm,
        src_v, dst_v, rows_v, acc, sem):
    cid = lax.axis_index("c")
    sid = lax.axis_index("s")
    wid = cid * NS + sid

    _zero_or_dump(sid, zeros_hbm, acc)
    plsc.subcore_barrier()

    n_mine = (N_CHUNKS - wid + NW - 1) // NW

    def body(j, carry):
      base = (wid + j * NW) * K
      pltpu.sync_copy(src_hbm.at[pl.ds(base, K)], src_v)
      pltpu.sync_copy(dst_hbm.at[pl.ds(base, K)], dst_v)
      pltpu.async_copy(y_hbm.at[src_v], rows_v, sem).wait()
      pltpu.sync_copy(rows_v, acc.at[dst_v], add=True)
      return carry

    lax.fori_loop(0, n_mine, body, 0)

    plsc.subcore_barrier()
    _zero_or_dump(sid, None, acc, out_ref=out_hbm, cid=cid)

  return k


def _bn(h, gamma, beta, eps=1e-5):
  m = jnp.mean(h, axis=0, keepdims=True)
  v = jnp.mean((h - m) ** 2, axis=0, keepdims=True)
  return gamma * (h - m) * jax.lax.rsqrt(v + eps) + beta


def _k0_body(x_ref, wl_ref, wr_ref, b_ref, y_ref, z_ref):
  x = x_ref[...]
  n = x.shape[0]
  y = jnp.dot(x, wl_ref[...], preferred_element_type=jnp.float32)
  pad1 = jnp.ones((n, 1), jnp.float32)
  pad0 = jnp.zeros((n, 63), jnp.float32)
  y_ref[...] = jnp.concatenate([y, pad1, pad0], axis=1)
  z = jnp.dot(x, wr_ref[...], preferred_element_type=jnp.float32) + b_ref[...]
  z_ref[...] = jnp.concatenate([z, pad1 * 0.0, pad0], axis=1)


def _k0b_body(s_ref, z_ref, g_ref, be_ref, h_ref, inv_ref):
  s = s_ref[0] + s_ref[1]
  cnt = s[:, 64:65]
  inv = 1.0 / jnp.maximum(cnt, 1.0)
  inv_ref[...] = jnp.broadcast_to(inv, inv_ref.shape)
  h = s * inv + z_ref[...]
  h_ref[...] = jnp.maximum(_bn(h, g_ref[...], be_ref[...]), 0.0)


def _mid_body(h_ref, s_ref, inv_ref, wl_ref, wr_ref, b_ref, g_ref, be_ref, o_ref):
  inv = inv_ref[:, 0:1]
  mean = (s_ref[0] + s_ref[1]) * inv
  h = (jnp.dot(mean, wl_ref[...], preferred_element_type=jnp.float32)
       + jnp.dot(h_ref[...], wr_ref[...], preferred_element_type=jnp.float32)
       + b_ref[...])
  o_ref[...] = jnp.maximum(_bn(h, g_ref[...], be_ref[...]), 0.0)


def _wide_body(h_ref, sa_ref, sb_ref, inv_ref, wla_ref, wlb_ref, wr_ref,
               b_ref, g_ref, be_ref, o_ref):
  inv = inv_ref[:, 0:1]
  mean_a = (sa_ref[0] + sa_ref[1]) * inv
  mean_b = (sb_ref[0] + sb_ref[1]) * inv
  h = (jnp.dot(mean_a, wla_ref[...], preferred_element_type=jnp.float32)
       + jnp.dot(mean_b, wlb_ref[...], preferred_element_type=jnp.float32)
       + jnp.dot(h_ref[...], wr_ref[...], preferred_element_type=jnp.float32)
       + b_ref[...])
  o_ref[...] = jnp.maximum(_bn(h, g_ref[...], be_ref[...]), 0.0)


def _last_body(h_ref, sa_ref, sb_ref, inv_ref, wla_ref, wlb_ref, wr_ref,
               b_ref, o_ref):
  inv = inv_ref[:, 0:1]
  mean_a = (sa_ref[0] + sa_ref[1]) * inv
  mean_b = (sb_ref[0] + sb_ref[1]) * inv
  o_ref[...] = (jnp.dot(mean_a, wla_ref[...], preferred_element_type=jnp.float32)
                + jnp.dot(mean_b, wlb_ref[...], preferred_element_type=jnp.float32)
                + jnp.dot(h_ref[...], wr_ref[...], preferred_element_type=jnp.float32)
                + b_ref[...])


def kernel(x, edge_index, params):
  n = x.shape[0]
  src = edge_index[0]
  dst = edge_index[1]
  f32 = jnp.float32

  zeros128 = jnp.zeros((ROWS_TILE, W), f32)
  scat = _make_feat_scatter()
  sds = jax.ShapeDtypeStruct

  def pad_rows(w, rows=W):
    return jnp.pad(w, ((0, rows - w.shape[0]), (0, 0)))

  def pad_cols(v, cols, value=0.0):
    return jnp.pad(v.reshape(1, -1), ((0, 0), (0, cols - v.shape[0])),
                   constant_values=value)

  y0, z0 = pl.pallas_call(
      _k0_body,
      out_shape=(sds((n, W), f32), sds((n, W), f32)),
  )(x, params["Wl0"], params["Wr0"], params["b0"].reshape(1, 64))
  s0 = scat(y0, src, dst, zeros128)
  h1, inv = pl.pallas_call(
      _k0b_body,
      out_shape=(sds((n, W), f32), sds((n, 16), f32)),
  )(s0, z0, pad_cols(params["gamma0"], W, value=1.0), pad_cols(params["beta0"], W))

  s1 = scat(h1, src, dst, zeros128)
  h2 = pl.pallas_call(
      _mid_body,
      out_shape=sds((n, 128), f32),
  )(h1, s1, inv, pad_rows(params["Wl1"]), pad_rows(params["Wr1"]),
    params["b1"].reshape(1, 128),
    params["gamma1"].reshape(1, 128), params["beta1"].reshape(1, 128))

  s2 = scat(h2, src, dst, zeros128)
  h3 = pl.pallas_call(
      _mid_body,
      out_shape=sds((n, 256), f32),
  )(h2, s2, inv, params["Wl2"], params["Wr2"], params["b2"].reshape(1, 256),
    params["gamma2"].reshape(1, 256), params["beta2"].reshape(1, 256))

  s3a = scat(h3[:, :128], src, dst, zeros128)
  s3b = scat(h3[:, 128:], src, dst, zeros128)
  h4 = pl.pallas_call(
      _wide_body,
      out_shape=sds((n, 256), f32),
  )(h3, s3a, s3b, inv, params["Wl3"][:128], params["Wl3"][128:],
    params["Wr3"], params["b3"].reshape(1, 256),
    params["gamma3"].reshape(1, 256), params["beta3"].reshape(1, 256))

  s4a = scat(h4[:, :128], src, dst, zeros128)
  s4b = scat(h4[:, 128:], src, dst, zeros128)
  mb = 2000
  out = pl.pallas_call(
      _last_body,
      grid=(n // mb,),
      in_specs=[
          pl.BlockSpec((mb, 256), lambda i: (i, 0)),
          pl.BlockSpec((2, mb, 128), lambda i: (0, i, 0)),
          pl.BlockSpec((2, mb, 128), lambda i: (0, i, 0)),
          pl.BlockSpec((mb, 16), lambda i: (i, 0)),
          pl.BlockSpec((128, 576), lambda i: (0, 0)),
          pl.BlockSpec((128, 576), lambda i: (0, 0)),
          pl.BlockSpec((256, 576), lambda i: (0, 0)),
          pl.BlockSpec((1, 576), lambda i: (0, 0)),
      ],
      out_specs=pl.BlockSpec((mb, 576), lambda i: (i, 0)),
      out_shape=sds((n, 576), f32),
  )(h4, s4a, s4b, inv, params["Wl4"][:128], params["Wl4"][128:],
    params["Wr4"], params["b4"].reshape(1, 576))
  return out
